# baseline (speedup 1.0000x reference)
import jax
import jax.numpy as jnp
from jax.experimental import pallas as pl

N_EXP = 64
K = 8
_BLK = 2048


def _router_block(x_ref, w_ref, idx_ref, val_ref):
    x = x_ref[...]
    w = w_ref[...]
    s = jax.lax.dot_general(
        x, w, dimension_numbers=(((1,), (0,)), ((), ())),
        preferred_element_type=jnp.float32,
    )
    b = s.shape[0]
    st = s.T
    lane_f = jax.lax.broadcasted_iota(
        jnp.int32, (N_EXP, b), 0).astype(jnp.float32)
    row = jax.lax.broadcasted_iota(jnp.int32, (K, b), 0)
    acc_i = jnp.zeros((K, b), dtype=jnp.float32)
    acc_v = jnp.zeros((K, b), dtype=jnp.float32)
    for k in range(K):
        m = jnp.max(st, axis=0, keepdims=True)
        cand = jnp.where(st == m, lane_f, jnp.float32(N_EXP))
        i = jnp.min(cand, axis=0, keepdims=True)
        acc_i = jnp.where(row == k, i, acc_i)
        acc_v = jnp.where(row == k, m, acc_v)
        st = jnp.where(cand == i, -jnp.inf, st)
    e = jnp.exp(acc_v - jax.lax.slice(acc_v, (0, 0), (1, b)))
    w8 = e / jnp.sum(e, axis=0, keepdims=True)
    idx_ref[...] = acc_i.astype(jnp.int32)
    val_ref[...] = w8


def kernel(x, W):
    n_tokens, d_model = x.shape
    grid = (n_tokens // _BLK,)
    xb = x.astype(jnp.bfloat16)
    Wt = W.T.astype(jnp.bfloat16)
    idx_t, w_t = pl.pallas_call(
        _router_block,
        grid=grid,
        in_specs=[
            pl.BlockSpec((_BLK, d_model), lambda i: (i, 0)),
            pl.BlockSpec((d_model, N_EXP), lambda i: (0, 0)),
        ],
        out_specs=[
            pl.BlockSpec((K, _BLK), lambda i: (0, i)),
            pl.BlockSpec((K, _BLK), lambda i: (0, i)),
        ],
        out_shape=[
            jax.ShapeDtypeStruct((K, n_tokens), jnp.int32),
            jax.ShapeDtypeStruct((K, n_tokens), jnp.float32),
        ],
    )(xb, Wt)
    return idx_t.T, w_t.T


# speedup vs baseline: 1.0041x; 1.0041x over previous
import jax
import jax.numpy as jnp
from jax.experimental import pallas as pl
from jax.experimental.pallas import tpu as pltpu

N_EXP = 64
K = 8
_BLK = 4096


def _router_block(x_ref, w_ref, idx_ref, val_ref):
    x = x_ref[...]
    w = w_ref[...]
    s = jax.lax.dot_general(
        x, w, dimension_numbers=(((1,), (0,)), ((), ())),
        preferred_element_type=jnp.float32,
    )
    b = s.shape[0]
    st = s.T
    lane_f = jax.lax.broadcasted_iota(
        jnp.int32, (N_EXP, b), 0).astype(jnp.float32)
    row = jax.lax.broadcasted_iota(jnp.int32, (K, b), 0)
    acc_i = jnp.zeros((K, b), dtype=jnp.float32)
    acc_v = jnp.zeros((K, b), dtype=jnp.float32)
    for k in range(K):
        m = jnp.max(st, axis=0, keepdims=True)
        cand = jnp.where(st == m, lane_f, jnp.float32(N_EXP))
        i = jnp.min(cand, axis=0, keepdims=True)
        acc_i = jnp.where(row == k, i, acc_i)
        acc_v = jnp.where(row == k, m, acc_v)
        st = jnp.where(cand == i, -jnp.inf, st)
    e = jnp.exp(acc_v - jax.lax.slice(acc_v, (0, 0), (1, b)))
    w8 = e / jnp.sum(e, axis=0, keepdims=True)
    idx_ref[...] = acc_i.astype(jnp.int32)
    val_ref[...] = w8


def kernel(x, W):
    n_tokens, d_model = x.shape
    grid = (n_tokens // _BLK,)
    xb = x.astype(jnp.bfloat16)
    Wt = W.T.astype(jnp.bfloat16)
    idx_t, w_t = pl.pallas_call(
        _router_block,
        grid=grid,
        in_specs=[
            pl.BlockSpec((_BLK, d_model), lambda i: (i, 0)),
            pl.BlockSpec((d_model, N_EXP), lambda i: (0, 0)),
        ],
        out_specs=[
            pl.BlockSpec((K, _BLK), lambda i: (0, i)),
            pl.BlockSpec((K, _BLK), lambda i: (0, i)),
        ],
        out_shape=[
            jax.ShapeDtypeStruct((K, n_tokens), jnp.int32),
            jax.ShapeDtypeStruct((K, n_tokens), jnp.float32),
        ],
    )(xb, Wt)
    return idx_t.T, w_t.T
